# Initial kernel scaffold; baseline (speedup 1.0000x reference)
import jax
import jax.numpy as jnp
from jax.experimental import pallas as pl


def kernel(x, edge_index, W0, b0, W1, b1, W2, b2, W3, b3, W4, b4, W5, b5, W6, b6, W7, b7):
    raise NotImplementedError("write your pallas kernel here")


# speedup vs baseline: 8.5907x; 8.5907x over previous
import functools

import jax
import jax.numpy as jnp
from jax import lax
from jax.experimental import pallas as pl
from jax.experimental.pallas import tpu as pltpu
from jax.experimental.pallas import tpu_sc as plsc

NC = 2
NS = 16
NW = NC * NS
CH = 128
N = 10000
AR = 10240
RPT = AR // NS
ZI = RPT // CH


def _make_agg(F, nch):
    mesh = plsc.VectorSubcoreMesh(core_axis_name="c", subcore_axis_name="s")

    @functools.partial(
        pl.kernel,
        out_type=jax.ShapeDtypeStruct((NC, AR, F), jnp.float32),
        mesh=mesh,
        compiler_params=pltpu.CompilerParams(use_tc_tiling_on_sc=False),
        scratch_types=[
            pltpu.VMEM((nch, CH), jnp.int32),
            pltpu.VMEM((nch, CH), jnp.int32),
            pltpu.VMEM((CH, F), jnp.float32),
            pltpu.VMEM_SHARED((AR, F), jnp.float32),
            pltpu.SemaphoreType.DMA,
        ],
    )
    def agg(g_hbm, src_hbm, dst_hbm, out_hbm, src_v, dst_v, rows_v, acc, sem):
        cid = lax.axis_index("c")
        sid = lax.axis_index("s")
        wid = sid * NC + cid
        pltpu.sync_copy(src_hbm.at[wid], src_v)
        pltpu.sync_copy(dst_hbm.at[wid], dst_v)

        zvec = jnp.zeros((16,), jnp.float32)

        def zrow(i, carry):
            for jj in range(F // 16):
                rows_v[i, pl.ds(jj * 16, 16)] = zvec
            return carry

        lax.fori_loop(0, CH, zrow, 0)
        r0 = sid * RPT
        for z in range(ZI):
            pltpu.sync_copy(rows_v, acc.at[pl.ds(r0 + z * CH, CH)])
        plsc.subcore_barrier()

        def body(j, carry):
            pltpu.async_copy(g_hbm.at[src_v.at[j]], rows_v, sem).wait()
            pltpu.sync_copy(rows_v, acc.at[dst_v.at[j]], add=True)
            return carry

        lax.fori_loop(0, nch, body, 0)
        plsc.subcore_barrier()
        pltpu.sync_copy(acc.at[pl.ds(r0, RPT)],
                        out_hbm.at[cid, pl.ds(r0, RPT)])

    return agg


def _make_deg(nch):
    F = 16
    mesh = plsc.VectorSubcoreMesh(core_axis_name="c", subcore_axis_name="s")

    @functools.partial(
        pl.kernel,
        out_type=jax.ShapeDtypeStruct((NC, AR, F), jnp.float32),
        mesh=mesh,
        compiler_params=pltpu.CompilerParams(use_tc_tiling_on_sc=False),
        scratch_types=[
            pltpu.VMEM((nch, CH), jnp.int32),
            pltpu.VMEM((CH, F), jnp.float32),
            pltpu.VMEM_SHARED((AR, F), jnp.float32),
        ],
    )
    def deg(dst_hbm, out_hbm, dst_v, rows_v, acc):
        cid = lax.axis_index("c")
        sid = lax.axis_index("s")
        wid = sid * NC + cid
        pltpu.sync_copy(dst_hbm.at[wid], dst_v)

        zvec = jnp.zeros((16,), jnp.float32)

        def zrow(i, carry):
            rows_v[i, pl.ds(0, 16)] = zvec
            return carry

        lax.fori_loop(0, CH, zrow, 0)
        r0 = sid * RPT
        for z in range(ZI):
            pltpu.sync_copy(rows_v, acc.at[pl.ds(r0 + z * CH, CH)])
        plsc.subcore_barrier()

        ovec = jnp.full((16,), 1.0, jnp.float32)

        def orow(i, carry):
            rows_v[i, pl.ds(0, 16)] = ovec
            return carry

        lax.fori_loop(0, CH, orow, 0)

        def body(j, carry):
            pltpu.sync_copy(rows_v, acc.at[dst_v.at[j]], add=True)
            return carry

        lax.fori_loop(0, nch, body, 0)
        plsc.subcore_barrier()
        pltpu.sync_copy(acc.at[pl.ds(r0, RPT)],
                        out_hbm.at[cid, pl.ds(r0, RPT)])

    return deg


def _dinv_from_deg(deg_acc):
    def body(deg_ref, out_ref):
        d = deg_ref[0, :, 0:1] + deg_ref[1, :, 0:1] + 1.0
        out_ref[...] = lax.rsqrt(jnp.maximum(d, 1e-12))

    return pl.pallas_call(
        body,
        out_shape=jax.ShapeDtypeStruct((AR, 1), jnp.float32),
    )(deg_acc)


def _tc_stage(g, dinv, acc=None, in_scale=True, b_pre=None, relu_pre=False,
              Wa=None, ba=None, relu_a=False, Wb=None, out_scale=True,
              R=1000):
    Fin = g.shape[1]
    Fout = Wb.shape[1] if Wb is not None else (
        Wa.shape[1] if Wa is not None else Fin)

    operands = []
    specs = []
    flags = dict(has_acc=acc is not None, has_bpre=b_pre is not None,
                 has_wa=Wa is not None, has_ba=ba is not None,
                 has_wb=Wb is not None)
    if acc is not None:
        operands.append(acc)
        specs.append(pl.BlockSpec((2, R, Fin), lambda i: (0, i, 0)))
    operands.append(g)
    specs.append(pl.BlockSpec((R, Fin), lambda i: (i, 0)))
    operands.append(dinv)
    specs.append(pl.BlockSpec((R, 1), lambda i: (i, 0)))
    if b_pre is not None:
        operands.append(b_pre.reshape(1, -1))
        specs.append(pl.BlockSpec((1, Fin), lambda i: (0, 0)))
    if Wa is not None:
        operands.append(Wa)
        specs.append(pl.BlockSpec(Wa.shape, lambda i: (0, 0)))
    if ba is not None:
        operands.append(ba.reshape(1, -1))
        specs.append(pl.BlockSpec((1, ba.shape[0]), lambda i: (0, 0)))
    if Wb is not None:
        operands.append(Wb)
        specs.append(pl.BlockSpec(Wb.shape, lambda i: (0, 0)))

    def body(*refs):
        it = iter(refs)
        acc_ref = next(it) if flags["has_acc"] else None
        g_ref = next(it)
        dinv_ref = next(it)
        bpre_ref = next(it) if flags["has_bpre"] else None
        wa_ref = next(it) if flags["has_wa"] else None
        ba_ref = next(it) if flags["has_ba"] else None
        wb_ref = next(it) if flags["has_wb"] else None
        out_ref = next(it)

        t = g_ref[...]
        if acc_ref is not None:
            t = t + acc_ref[0] + acc_ref[1]
        dv = dinv_ref[...]
        if in_scale:
            t = t * dv
        if bpre_ref is not None:
            t = t + bpre_ref[...]
            if relu_pre:
                t = jnp.maximum(t, 0.0)
        if wa_ref is not None:
            t = jnp.dot(t, wa_ref[...], preferred_element_type=jnp.float32)
            if ba_ref is not None:
                t = t + ba_ref[...]
            if relu_a:
                t = jnp.maximum(t, 0.0)
        if wb_ref is not None:
            t = jnp.dot(t, wb_ref[...], preferred_element_type=jnp.float32)
        if out_scale:
            t = t * dv
        out_ref[...] = t

    return pl.pallas_call(
        body,
        grid=(N // R,),
        in_specs=specs,
        out_specs=pl.BlockSpec((R, Fout), lambda i: (i, 0)),
        out_shape=jax.ShapeDtypeStruct((N, Fout), jnp.float32),
    )(*operands)


def _agg_call(g, src_slab, dst_slab, nch):
    F = g.shape[1]
    if F <= CH:
        return _make_agg(F, nch)(g, src_slab, dst_slab)
    parts = [
        _make_agg(CH, nch)(
            lax.slice_in_dim(g, c * CH, (c + 1) * CH, axis=1),
            src_slab, dst_slab)
        for c in range(F // CH)
    ]
    return jnp.concatenate(parts, axis=2)


def kernel(x, edge_index, W0, b0, W1, b1, W2, b2, W3, b3, W4, b4, W5, b5,
           W6, b6, W7, b7):
    E = edge_index.shape[1]
    nch = -(-E // (NW * CH))
    Epad = NW * nch * CH
    src_p = jnp.concatenate(
        [edge_index[0], jnp.zeros((Epad - E,), jnp.int32)])
    dst_p = jnp.concatenate(
        [edge_index[1], jnp.full((Epad - E,), N, jnp.int32)])
    src_slab = src_p.reshape(NW, nch, CH)
    dst_slab = dst_p.reshape(NW, nch, CH)

    deg_acc = _make_deg(nch)(dst_slab)
    dinv = _dinv_from_deg(deg_acc)

    g = _tc_stage(x, dinv, in_scale=False, Wa=W0)
    acc = _agg_call(g, src_slab, dst_slab, nch)
    for b_i, W_next in ((b0, W1), (b1, W2)):
        g = _tc_stage(g, dinv, acc=acc, b_pre=b_i, relu_pre=True, Wa=W_next)
        acc = _agg_call(g, src_slab, dst_slab, nch)
    g = _tc_stage(g, dinv, acc=acc, b_pre=b2, relu_pre=True)
    acc = _agg_call(g, src_slab, dst_slab, nch)
    g = _tc_stage(g, dinv, acc=acc, Wa=W3, ba=b3, relu_a=True)
    acc = _agg_call(g, src_slab, dst_slab, nch)
    g = _tc_stage(g, dinv, acc=acc, Wa=W4, ba=b4, relu_a=True, Wb=W5)
    acc = _agg_call(g, src_slab, dst_slab, nch)
    for b_i, W_next in ((b5, W6), (b6, W7)):
        g = _tc_stage(g, dinv, acc=acc, b_pre=b_i, relu_pre=True, Wa=W_next)
        acc = _agg_call(g, src_slab, dst_slab, nch)
    return _tc_stage(g, dinv, acc=acc, b_pre=b7, in_scale=True,
                     out_scale=False)
